# baseline (speedup 1.0000x reference)
import jax
import jax.numpy as jnp
from jax import lax
from jax.experimental import pallas as pl
from jax.experimental.pallas import tpu as pltpu

BM = 512
BN = 4096
SUB = 1024
NSUB = BN // SUB
LANES = 128
NCH = SUB // LANES


def _argmin_kernel(x_ref, c_ref, cn_ref, out_ref, best_val, best_idx):
    j = pl.program_id(0)
    i = pl.program_id(1)
    nj = pl.num_programs(0)

    xb = x_ref[...]
    xsq = jnp.sum(xb * xb, axis=1, keepdims=True)
    xb2 = xb * -2.0

    xb2h = xb2.astype(jnp.bfloat16)

    m = None
    a = None
    for n in range(NSUB):
        acc2 = jnp.dot(
            xb2h,
            c_ref[:, n * SUB:(n + 1) * SUB].astype(jnp.bfloat16),
            preferred_element_type=jnp.float32,
        )
        for t in range(NCH):
            k = n * NCH + t
            sl = slice(t * LANES, (t + 1) * LANES)
            ch = (xsq + acc2[:, sl]) + cn_ref[:, n * SUB:(n + 1) * SUB][:, sl]
            if m is None:
                m, a = ch, jnp.zeros_like(ch)
            else:
                upd = ch < m
                m = jnp.minimum(m, ch)
                a = jnp.where(upd, float(k), a)

    lane = lax.broadcasted_iota(jnp.int32, m.shape, 1).astype(jnp.float32)
    col = a * float(LANES) + lane
    bv = jnp.min(m, axis=1, keepdims=True)
    bi = jnp.min(jnp.where(m == bv, col, float(BN)), axis=1, keepdims=True)
    bi = bi + (j * BN).astype(jnp.float32)

    rows = pl.ds(i * BM, BM)

    @pl.when(j == 0)
    def _():
        best_val[rows, :] = bv
        best_idx[rows, :] = bi

    @pl.when(j > 0)
    def _():
        better = bv < best_val[rows, :]
        best_val[rows, :] = jnp.where(better, bv, best_val[rows, :])
        best_idx[rows, :] = jnp.where(better, bi, best_idx[rows, :])

    @pl.when(j == nj - 1)
    def _():
        out_ref[...] = best_idx[rows, :].astype(jnp.int32)


def kernel(x, C, Cnorm):
    B, T, D = x.shape
    K = C.shape[1]
    M = B * T
    x2 = x.reshape(M, D)

    grid = (K // BN, M // BM)
    out = pl.pallas_call(
        _argmin_kernel,
        grid=grid,
        in_specs=[
            pl.BlockSpec((BM, D), lambda j, i: (i, 0)),
            pl.BlockSpec((D, BN), lambda j, i: (0, j)),
            pl.BlockSpec((1, BN), lambda j, i: (0, j)),
        ],
        out_specs=pl.BlockSpec((BM, 1), lambda j, i: (i, 0)),
        out_shape=jax.ShapeDtypeStruct((M, 1), jnp.int32),
        scratch_shapes=[
            pltpu.VMEM((M, 1), jnp.float32),
            pltpu.VMEM((M, 1), jnp.float32),
        ],
        compiler_params=pltpu.CompilerParams(
            dimension_semantics=("arbitrary", "arbitrary"),
            vmem_limit_bytes=60000 * 1024,
        ),
    )(x2, C, Cnorm)
    return out.reshape(B, T, 1)


# speedup vs baseline: 1.1057x; 1.1057x over previous
import jax
import jax.numpy as jnp
from jax import lax
from jax.experimental import pallas as pl
from jax.experimental.pallas import tpu as pltpu

BM = 1024
BN = 8192
SUB = 1024
NSUB = BN // SUB
LANES = 128
NCH = SUB // LANES


def _argmin_kernel(x_ref, c_ref, cn_ref, out_ref, best_val, best_idx):
    j = pl.program_id(0)
    i = pl.program_id(1)
    nj = pl.num_programs(0)

    xb = x_ref[...]
    xsq = jnp.sum(xb * xb, axis=1, keepdims=True)
    xb2 = xb * -2.0

    xb2h = xb2.astype(jnp.bfloat16)

    m = None
    a = None
    for n in range(NSUB):
        acc2 = jnp.dot(
            xb2h,
            c_ref[:, n * SUB:(n + 1) * SUB].astype(jnp.bfloat16),
            preferred_element_type=jnp.float32,
        )
        for t in range(NCH):
            k = n * NCH + t
            sl = slice(t * LANES, (t + 1) * LANES)
            ch = (xsq + acc2[:, sl]) + cn_ref[:, n * SUB:(n + 1) * SUB][:, sl]
            if m is None:
                m, a = ch, jnp.zeros_like(ch)
            else:
                upd = ch < m
                m = jnp.minimum(m, ch)
                a = jnp.where(upd, float(k), a)

    lane = lax.broadcasted_iota(jnp.int32, m.shape, 1).astype(jnp.float32)
    col = a * float(LANES) + lane
    bv = jnp.min(m, axis=1, keepdims=True)
    bi = jnp.min(jnp.where(m == bv, col, float(BN)), axis=1, keepdims=True)
    bi = bi + (j * BN).astype(jnp.float32)

    rows = pl.ds(i * BM, BM)

    @pl.when(j == 0)
    def _():
        best_val[rows, :] = bv
        best_idx[rows, :] = bi

    @pl.when(j > 0)
    def _():
        better = bv < best_val[rows, :]
        best_val[rows, :] = jnp.where(better, bv, best_val[rows, :])
        best_idx[rows, :] = jnp.where(better, bi, best_idx[rows, :])

    @pl.when(j == nj - 1)
    def _():
        out_ref[...] = best_idx[rows, :].astype(jnp.int32)


def kernel(x, C, Cnorm):
    B, T, D = x.shape
    K = C.shape[1]
    M = B * T
    x2 = x.reshape(M, D)

    grid = (K // BN, M // BM)
    out = pl.pallas_call(
        _argmin_kernel,
        grid=grid,
        in_specs=[
            pl.BlockSpec((BM, D), lambda j, i: (i, 0)),
            pl.BlockSpec((D, BN), lambda j, i: (0, j)),
            pl.BlockSpec((1, BN), lambda j, i: (0, j)),
        ],
        out_specs=pl.BlockSpec((BM, 1), lambda j, i: (i, 0)),
        out_shape=jax.ShapeDtypeStruct((M, 1), jnp.int32),
        scratch_shapes=[
            pltpu.VMEM((M, 1), jnp.float32),
            pltpu.VMEM((M, 1), jnp.float32),
        ],
        compiler_params=pltpu.CompilerParams(
            dimension_semantics=("arbitrary", "arbitrary"),
            vmem_limit_bytes=60000 * 1024,
        ),
    )(x2, C, Cnorm)
    return out.reshape(B, T, 1)


# speedup vs baseline: 1.1122x; 1.0058x over previous
import jax
import jax.numpy as jnp
from jax import lax
from jax.experimental import pallas as pl
from jax.experimental.pallas import tpu as pltpu

BM = 1024
BN = 8192
SUB = 256
NSUB = BN // SUB
LANES = 128
NCH = SUB // LANES


def _argmin_kernel(x_ref, c_ref, cn_ref, out_ref, best_val, best_idx):
    j = pl.program_id(0)
    i = pl.program_id(1)
    nj = pl.num_programs(0)

    xb = x_ref[...]
    xsq = jnp.sum(xb * xb, axis=1, keepdims=True)
    xb2 = xb * -2.0

    xb2h = xb2.astype(jnp.bfloat16)

    m = None
    a = None
    for n in range(NSUB):
        acc2 = jnp.dot(
            xb2h,
            c_ref[:, n * SUB:(n + 1) * SUB].astype(jnp.bfloat16),
            preferred_element_type=jnp.float32,
        )
        for t in range(NCH):
            k = n * NCH + t
            sl = slice(t * LANES, (t + 1) * LANES)
            ch = (xsq + acc2[:, sl]) + cn_ref[:, n * SUB:(n + 1) * SUB][:, sl]
            if m is None:
                m, a = ch, jnp.zeros_like(ch)
            else:
                upd = ch < m
                m = jnp.minimum(m, ch)
                a = jnp.where(upd, float(k), a)

    lane = lax.broadcasted_iota(jnp.int32, m.shape, 1).astype(jnp.float32)
    col = a * float(LANES) + lane
    bv = jnp.min(m, axis=1, keepdims=True)
    bi = jnp.min(jnp.where(m == bv, col, float(BN)), axis=1, keepdims=True)
    bi = bi + (j * BN).astype(jnp.float32)

    rows = pl.ds(i * BM, BM)

    @pl.when(j == 0)
    def _():
        best_val[rows, :] = bv
        best_idx[rows, :] = bi

    @pl.when(j > 0)
    def _():
        better = bv < best_val[rows, :]
        best_val[rows, :] = jnp.where(better, bv, best_val[rows, :])
        best_idx[rows, :] = jnp.where(better, bi, best_idx[rows, :])

    @pl.when(j == nj - 1)
    def _():
        out_ref[...] = best_idx[rows, :].astype(jnp.int32)


def kernel(x, C, Cnorm):
    B, T, D = x.shape
    K = C.shape[1]
    M = B * T
    x2 = x.reshape(M, D)

    grid = (K // BN, M // BM)
    out = pl.pallas_call(
        _argmin_kernel,
        grid=grid,
        in_specs=[
            pl.BlockSpec((BM, D), lambda j, i: (i, 0)),
            pl.BlockSpec((D, BN), lambda j, i: (0, j)),
            pl.BlockSpec((1, BN), lambda j, i: (0, j)),
        ],
        out_specs=pl.BlockSpec((BM, 1), lambda j, i: (i, 0)),
        out_shape=jax.ShapeDtypeStruct((M, 1), jnp.int32),
        scratch_shapes=[
            pltpu.VMEM((M, 1), jnp.float32),
            pltpu.VMEM((M, 1), jnp.float32),
        ],
        compiler_params=pltpu.CompilerParams(
            dimension_semantics=("arbitrary", "arbitrary"),
            vmem_limit_bytes=60000 * 1024,
        ),
    )(x2, C, Cnorm)
    return out.reshape(B, T, 1)
